# Initial kernel scaffold; baseline (speedup 1.0000x reference)
import jax
import jax.numpy as jnp
from jax.experimental import pallas as pl


def kernel(x, weight):
    raise NotImplementedError("write your pallas kernel here")


# speedup vs baseline: 1.0936x; 1.0936x over previous
import functools

import jax
import jax.numpy as jnp
from jax import lax
from jax.experimental import pallas as pl
from jax.experimental.pallas import tpu as pltpu
from jax.experimental.pallas import tpu_sc as plsc

NUM_EMBEDDINGS = 1000000
EMBEDDING_DIM = 32
BATCH = 16384
HIST = 50

_INFO = plsc.get_sparse_core_info()
_NC = _INFO.num_cores
_NS = _INFO.num_subcores
_NW = _NC * _NS
_TOTAL = BATCH * HIST
_PER_W = _TOTAL // _NW
_CHUNK = 1024
_NCHUNKS = _PER_W // _CHUNK


def _gather_body(table_hbm, idx_hbm, out_hbm, idx_v, rows_v, sem):
    wid = lax.axis_index("s") * _NC + lax.axis_index("c")
    base = wid * _PER_W

    def chunk(i, carry):
        off = base + i * _CHUNK
        pltpu.sync_copy(idx_hbm.at[pl.ds(off, _CHUNK)], idx_v)
        pltpu.async_copy(table_hbm.at[idx_v], rows_v, sem).wait()
        pltpu.sync_copy(rows_v, out_hbm.at[pl.ds(off, _CHUNK)])
        return carry

    lax.fori_loop(0, _NCHUNKS, chunk, 0)


@jax.jit
def _lookup(x_flat, weight):
    mesh = plsc.VectorSubcoreMesh(core_axis_name="c", subcore_axis_name="s")
    f = functools.partial(
        pl.kernel,
        out_type=jax.ShapeDtypeStruct((_TOTAL, EMBEDDING_DIM), jnp.float32),
        mesh=mesh,
        scratch_types=[
            pltpu.VMEM((_CHUNK,), jnp.int32),
            pltpu.VMEM((_CHUNK, EMBEDDING_DIM), jnp.float32),
            pltpu.SemaphoreType.DMA,
        ],
        compiler_params=pltpu.CompilerParams(use_tc_tiling_on_sc=False),
    )(_gather_body)
    return f(weight, x_flat)


def kernel(x, weight):
    x_flat = x.reshape(-1).astype(jnp.int32)
    out = _lookup(x_flat, weight)
    return out.reshape(BATCH, HIST, EMBEDDING_DIM)


# speedup vs baseline: 1.6471x; 1.5062x over previous
import functools

import jax
import jax.numpy as jnp
from jax import lax
from jax.experimental import pallas as pl
from jax.experimental.pallas import tpu as pltpu
from jax.experimental.pallas import tpu_sc as plsc

NUM_EMBEDDINGS = 1000000
EMBEDDING_DIM = 32
BATCH = 16384
HIST = 50

_INFO = plsc.get_sparse_core_info()
_NC = _INFO.num_cores
_NS = _INFO.num_subcores
_NW = _NC * _NS
_TOTAL = BATCH * HIST
_PER_W = _TOTAL // _NW
_C = 512
_BT = _C // 128
_DT = EMBEDDING_DIM // 8
_UNITS = _PER_W // _C


def _gather_body(table_hbm, idx_hbm, out_hbm,
                 idx_v, rows0, rows1, txp0, txp1,
                 isem, gsem0, gsem1, osem0, osem1):
    wid = lax.axis_index("s") * _NC + lax.axis_index("c")
    base = wid * _PER_W
    pltpu.async_copy(idx_hbm.at[pl.ds(base, _PER_W)], idx_v, isem).wait()

    iota16 = lax.iota(jnp.int32, 16)
    bufs = ((rows0, txp0, gsem0, osem0), (rows1, txp1, gsem1, osem1))

    def start_gather(u, rows, gsem):
        pltpu.async_copy(table_hbm.at[idx_v.at[pl.ds(u * _C, _C)]], rows, gsem)

    def out_slice(u):
        g = base + u * _C
        h = g // BATCH
        bc = (g % BATCH) // _C
        return out_hbm.at[h, :, pl.ds(bc * _BT, _BT)]

    def transpose(rows, txp):
        def col(k, carry):
            dt = k >> 3
            dr = k & 7
            colv = jnp.zeros((16,), jnp.int32) + (dt * 8 + dr)
            for btl in range(_BT):
                rb = btl * 128
                for brg in range(8):
                    ridx = iota16 + (rb + brg * 16)
                    txp[dt, btl, dr, pl.ds(brg * 16, 16)] = (
                        plsc.load_gather(rows, [ridx, colv]))
            return carry
        lax.fori_loop(0, _DT * 8, col, 0)

    start_gather(0, rows0, gsem0)
    start_gather(1, rows1, gsem1)

    def pair(p, carry):
        for b, (rows, txp, gsem, osem) in enumerate(bufs):
            u = 2 * p + b
            pltpu.make_async_copy(
                table_hbm.at[idx_v.at[pl.ds(u * _C, _C)]], rows, gsem).wait()

            @pl.when(p > 0)
            def _():
                pltpu.make_async_copy(txp, out_slice(u - 2), osem).wait()

            transpose(rows, txp)
            pltpu.async_copy(txp, out_slice(u), osem)

            @pl.when(u + 2 < _UNITS)
            def _():
                start_gather(u + 2, rows, gsem)
        return carry

    lax.fori_loop(0, _UNITS // 2, pair, 0)

    pltpu.make_async_copy(txp0, out_slice(_UNITS - 2), osem0).wait()
    pltpu.make_async_copy(txp1, out_slice(_UNITS - 1), osem1).wait()


@jax.jit
def _lookup(x_flat, weight):
    mesh = plsc.VectorSubcoreMesh(core_axis_name="c", subcore_axis_name="s")
    f = functools.partial(
        pl.kernel,
        out_type=jax.ShapeDtypeStruct(
            (HIST, _DT, BATCH // 128, 8, 128), jnp.float32),
        mesh=mesh,
        scratch_types=[
            pltpu.VMEM((_PER_W,), jnp.int32),
            pltpu.VMEM((_C, EMBEDDING_DIM), jnp.float32),
            pltpu.VMEM((_C, EMBEDDING_DIM), jnp.float32),
            pltpu.VMEM((_DT, _BT, 8, 128), jnp.float32),
            pltpu.VMEM((_DT, _BT, 8, 128), jnp.float32),
            pltpu.SemaphoreType.DMA,
            pltpu.SemaphoreType.DMA,
            pltpu.SemaphoreType.DMA,
            pltpu.SemaphoreType.DMA,
            pltpu.SemaphoreType.DMA,
        ],
        compiler_params=pltpu.CompilerParams(
            use_tc_tiling_on_sc=False, needs_layout_passes=False),
    )(_gather_body)
    return f(weight, x_flat)


def kernel(x, weight):
    x_flat = jnp.swapaxes(x, 0, 1).reshape(-1).astype(jnp.int32)
    out5 = _lookup(x_flat, weight)
    return out5.transpose(2, 4, 0, 1, 3).reshape(BATCH, HIST, EMBEDDING_DIM)


# speedup vs baseline: 2.0990x; 1.2743x over previous
import functools

import jax
import jax.numpy as jnp
from jax import lax
from jax.experimental import pallas as pl
from jax.experimental.pallas import tpu as pltpu
from jax.experimental.pallas import tpu_sc as plsc

NUM_EMBEDDINGS = 1000000
EMBEDDING_DIM = 32
BATCH = 16384
HIST = 50

_INFO = plsc.get_sparse_core_info()
_NC = _INFO.num_cores
_NS = _INFO.num_subcores
_NW = _NC * _NS
_TOTAL = BATCH * HIST
_PER_W = _TOTAL // _NW
_C = 512
_BT = _C // 128
_DT = EMBEDDING_DIM // 8
_UNITS = _PER_W // _C


def _gather_body(table_hbm, idx_hbm, out_hbm,
                 idx_v, rows0, rows1, txp0, txp1,
                 isem, gsem0, gsem1, osem0, osem1):
    wid = lax.axis_index("s") * _NC + lax.axis_index("c")
    base = wid * _PER_W
    pltpu.async_copy(idx_hbm.at[pl.ds(base, _PER_W)], idx_v, isem).wait()

    iota16 = lax.iota(jnp.int32, 16)
    bufs = ((rows0, txp0, gsem0, osem0), (rows1, txp1, gsem1, osem1))

    def start_gather(u, rows, gsem):
        pltpu.async_copy(table_hbm.at[idx_v.at[pl.ds(u * _C, _C)]], rows, gsem)

    def out_slice(u):
        g = base + u * _C
        h = g // BATCH
        bc = (g % BATCH) // _C
        return out_hbm.at[h, :, pl.ds(bc * _BT, _BT)]

    def transpose(rows, txp):
        def col(k):
            dt = k >> 3
            dr = k & 7
            colv = jnp.zeros((16,), jnp.int32) + (dt * 8 + dr)
            for btl in range(_BT):
                rb = btl * 128
                for brg in range(8):
                    ridx = iota16 + (rb + brg * 16)
                    txp[dt, btl, dr, pl.ds(brg * 16, 16)] = (
                        plsc.load_gather(rows, [ridx, colv]))
        plsc.parallel_loop(0, _DT * 8, 1, unroll=2)(col)

    start_gather(0, rows0, gsem0)
    start_gather(1, rows1, gsem1)

    def pair(p, carry):
        for b, (rows, txp, gsem, osem) in enumerate(bufs):
            u = 2 * p + b
            pltpu.make_async_copy(
                table_hbm.at[idx_v.at[pl.ds(u * _C, _C)]], rows, gsem).wait()

            @pl.when(p > 0)
            def _():
                pltpu.make_async_copy(txp, out_slice(u - 2), osem).wait()

            transpose(rows, txp)
            pltpu.async_copy(txp, out_slice(u), osem)

            @pl.when(u + 2 < _UNITS)
            def _():
                start_gather(u + 2, rows, gsem)
        return carry

    lax.fori_loop(0, _UNITS // 2, pair, 0)

    pltpu.make_async_copy(txp0, out_slice(_UNITS - 2), osem0).wait()
    pltpu.make_async_copy(txp1, out_slice(_UNITS - 1), osem1).wait()


@jax.jit
def _lookup(x_flat, weight):
    mesh = plsc.VectorSubcoreMesh(core_axis_name="c", subcore_axis_name="s")
    f = functools.partial(
        pl.kernel,
        out_type=jax.ShapeDtypeStruct(
            (HIST, _DT, BATCH // 128, 8, 128), jnp.float32),
        mesh=mesh,
        scratch_types=[
            pltpu.VMEM((_PER_W,), jnp.int32),
            pltpu.VMEM((_C, EMBEDDING_DIM), jnp.float32),
            pltpu.VMEM((_C, EMBEDDING_DIM), jnp.float32),
            pltpu.VMEM((_DT, _BT, 8, 128), jnp.float32),
            pltpu.VMEM((_DT, _BT, 8, 128), jnp.float32),
            pltpu.SemaphoreType.DMA,
            pltpu.SemaphoreType.DMA,
            pltpu.SemaphoreType.DMA,
            pltpu.SemaphoreType.DMA,
            pltpu.SemaphoreType.DMA,
        ],
        compiler_params=pltpu.CompilerParams(
            use_tc_tiling_on_sc=False, needs_layout_passes=False),
    )(_gather_body)
    return f(weight, x_flat)


def kernel(x, weight):
    x_flat = jnp.swapaxes(x, 0, 1).reshape(-1).astype(jnp.int32)
    out5 = _lookup(x_flat, weight)
    return out5.transpose(2, 4, 0, 1, 3).reshape(BATCH, HIST, EMBEDDING_DIM)


# speedup vs baseline: 2.1123x; 1.0064x over previous
import functools

import jax
import jax.numpy as jnp
from jax import lax
from jax.experimental import pallas as pl
from jax.experimental.pallas import tpu as pltpu
from jax.experimental.pallas import tpu_sc as plsc

NUM_EMBEDDINGS = 1000000
EMBEDDING_DIM = 32
BATCH = 16384
HIST = 50

_INFO = plsc.get_sparse_core_info()
_NC = _INFO.num_cores
_NS = _INFO.num_subcores
_NW = _NC * _NS
_TOTAL = BATCH * HIST
_PER_W = _TOTAL // _NW
_C = 512
_BT = _C // 128
_DT = EMBEDDING_DIM // 8
_UNITS = _PER_W // _C


def _gather_body(table_hbm, idx_hbm, out_hbm,
                 idx_v, rows0, rows1, txp0, txp1,
                 isem, gsem0, gsem1, osem0, osem1):
    wid = lax.axis_index("s") * _NC + lax.axis_index("c")
    base = wid * _PER_W
    pltpu.async_copy(idx_hbm.at[pl.ds(base, _PER_W)], idx_v, isem).wait()

    iota16 = lax.iota(jnp.int32, 16)
    ridxs = tuple(
        tuple(iota16 + (btl * 128 + brg * 16) for brg in range(8))
        for btl in range(_BT))
    bufs = ((rows0, txp0, gsem0, osem0), (rows1, txp1, gsem1, osem1))

    def start_gather(u, rows, gsem):
        pltpu.async_copy(table_hbm.at[idx_v.at[pl.ds(u * _C, _C)]], rows, gsem)

    def out_slice(u):
        g = base + u * _C
        h = g // BATCH
        bc = (g % BATCH) // _C
        return out_hbm.at[h, :, pl.ds(bc * _BT, _BT)]

    def transpose(rows, txp):
        def col(k):
            dt = k >> 3
            dr = k & 7
            colv = jnp.zeros((16,), jnp.int32) + (dt * 8 + dr)
            for btl in range(_BT):
                for brg in range(8):
                    txp[dt, btl, dr, pl.ds(brg * 16, 16)] = (
                        plsc.load_gather(rows, [ridxs[btl][brg], colv]))
        plsc.parallel_loop(0, _DT * 8, 1, unroll=4)(col)

    start_gather(0, rows0, gsem0)
    start_gather(1, rows1, gsem1)

    def pair(p, carry):
        for b, (rows, txp, gsem, osem) in enumerate(bufs):
            u = 2 * p + b
            pltpu.make_async_copy(
                table_hbm.at[idx_v.at[pl.ds(u * _C, _C)]], rows, gsem).wait()

            @pl.when(p > 0)
            def _():
                pltpu.make_async_copy(txp, out_slice(u - 2), osem).wait()

            transpose(rows, txp)
            pltpu.async_copy(txp, out_slice(u), osem)

            @pl.when(u + 2 < _UNITS)
            def _():
                start_gather(u + 2, rows, gsem)
        return carry

    lax.fori_loop(0, _UNITS // 2, pair, 0)

    pltpu.make_async_copy(txp0, out_slice(_UNITS - 2), osem0).wait()
    pltpu.make_async_copy(txp1, out_slice(_UNITS - 1), osem1).wait()


@jax.jit
def _lookup(x_flat, weight):
    mesh = plsc.VectorSubcoreMesh(core_axis_name="c", subcore_axis_name="s")
    f = functools.partial(
        pl.kernel,
        out_type=jax.ShapeDtypeStruct(
            (HIST, _DT, BATCH // 128, 8, 128), jnp.float32),
        mesh=mesh,
        scratch_types=[
            pltpu.VMEM((_PER_W,), jnp.int32),
            pltpu.VMEM((_C, EMBEDDING_DIM), jnp.float32),
            pltpu.VMEM((_C, EMBEDDING_DIM), jnp.float32),
            pltpu.VMEM((_DT, _BT, 8, 128), jnp.float32),
            pltpu.VMEM((_DT, _BT, 8, 128), jnp.float32),
            pltpu.SemaphoreType.DMA,
            pltpu.SemaphoreType.DMA,
            pltpu.SemaphoreType.DMA,
            pltpu.SemaphoreType.DMA,
            pltpu.SemaphoreType.DMA,
        ],
        compiler_params=pltpu.CompilerParams(
            use_tc_tiling_on_sc=False, needs_layout_passes=False),
    )(_gather_body)
    return f(weight, x_flat)


def kernel(x, weight):
    x_flat = jnp.swapaxes(x, 0, 1).reshape(-1).astype(jnp.int32)
    out5 = _lookup(x_flat, weight)
    return out5.transpose(2, 4, 0, 1, 3).reshape(BATCH, HIST, EMBEDDING_DIM)


# speedup vs baseline: 2.8497x; 1.3491x over previous
import functools

import jax
import jax.numpy as jnp
from jax import lax
from jax.experimental import pallas as pl
from jax.experimental.pallas import tpu as pltpu
from jax.experimental.pallas import tpu_sc as plsc

NUM_EMBEDDINGS = 1000000
EMBEDDING_DIM = 32
BATCH = 16384
HIST = 50

_INFO = plsc.get_sparse_core_info()
_NC = _INFO.num_cores
_NS = _INFO.num_subcores
_NW = _NC * _NS
_TOTAL = BATCH * HIST
_PER_W = _TOTAL // _NW
_C = 512
_BT = _C // 128
_DT = EMBEDDING_DIM // 8
_UNITS = _PER_W // _C


def _gather_body(table_hbm, idx_hbm, out_hbm,
                 idx_v, rows0, rows1, rowsp, txp0, txp1,
                 isem, gsem0, gsem1, osem0, osem1):
    wid = lax.axis_index("s") * _NC + lax.axis_index("c")
    base = wid * _PER_W
    pltpu.async_copy(idx_hbm.at[pl.ds(base, _PER_W)], idx_v, isem).wait()

    iota16 = lax.iota(jnp.int32, 16)
    ridxs = tuple(
        tuple(iota16 + (btl * 128 + brg * 16) for brg in range(8))
        for btl in range(_BT))
    bufs = ((rows0, txp0, gsem0, osem0),
            (rows1, txp1, gsem1, osem1))

    def start_gather(u, rows, gsem):
        pltpu.async_copy(table_hbm.at[idx_v.at[pl.ds(u * _C, _C)]], rows, gsem)

    def gather_wait(u, rows, gsem):
        pltpu.make_async_copy(
            table_hbm.at[idx_v.at[pl.ds(u * _C, _C)]], rows, gsem).wait()

    def out_slice(u):
        g = base + u * _C
        h = g // BATCH
        bc = (g % BATCH) // _C
        return out_hbm.at[h, :, pl.ds(bc * _BT, _BT)]

    def transpose(rowsp, txp):
        def col(k):
            dt = k >> 3
            dr = k & 7
            colv = jnp.zeros((16,), jnp.int32) + (dt * 8 + dr)
            for btl in range(_BT):
                for brg in range(8):
                    txp[dt, btl, dr, pl.ds(brg * 16, 16)] = (
                        plsc.load_gather(rowsp, [ridxs[btl][brg], colv]))
        plsc.parallel_loop(0, _DT * 8, 1, unroll=4)(col)

    def repitch(rows, rowsp):
        def rp(r):
            rowsp[r, pl.ds(0, 16)] = rows[r, pl.ds(0, 16)]
            rowsp[r, pl.ds(16, 16)] = rows[r, pl.ds(16, 16)]
        plsc.parallel_loop(0, _C, 1, unroll=8)(rp)

    start_gather(0, rows0, gsem0)
    start_gather(1, rows1, gsem1)

    def pair(p, carry):
        for b, (rows, txp, gsem, osem) in enumerate(bufs):
            u = 2 * p + b
            gather_wait(u, rows, gsem)
            repitch(rows, rowsp)

            @pl.when(u + 2 < _UNITS)
            def _():
                start_gather(u + 2, rows, gsem)

            @pl.when(p > 0)
            def _():
                pltpu.make_async_copy(txp, out_slice(u - 2), osem).wait()

            transpose(rowsp, txp)
            pltpu.async_copy(txp, out_slice(u), osem)
        return carry

    lax.fori_loop(0, _UNITS // 2, pair, 0)

    pltpu.make_async_copy(txp0, out_slice(_UNITS - 2), osem0).wait()
    pltpu.make_async_copy(txp1, out_slice(_UNITS - 1), osem1).wait()


@jax.jit
def _lookup(x_flat, weight):
    mesh = plsc.VectorSubcoreMesh(core_axis_name="c", subcore_axis_name="s")
    f = functools.partial(
        pl.kernel,
        out_type=jax.ShapeDtypeStruct(
            (HIST, _DT, BATCH // 128, 8, 128), jnp.float32),
        mesh=mesh,
        scratch_types=[
            pltpu.VMEM((_PER_W,), jnp.int32),
            pltpu.VMEM((_C, EMBEDDING_DIM), jnp.float32),
            pltpu.VMEM((_C, EMBEDDING_DIM), jnp.float32),
            pltpu.VMEM((_C, EMBEDDING_DIM + 1), jnp.float32),
            pltpu.VMEM((_DT, _BT, 8, 128), jnp.float32),
            pltpu.VMEM((_DT, _BT, 8, 128), jnp.float32),
            pltpu.SemaphoreType.DMA,
            pltpu.SemaphoreType.DMA,
            pltpu.SemaphoreType.DMA,
            pltpu.SemaphoreType.DMA,
            pltpu.SemaphoreType.DMA,
        ],
        compiler_params=pltpu.CompilerParams(
            use_tc_tiling_on_sc=False, needs_layout_passes=False),
    )(_gather_body)
    return f(weight, x_flat)


def kernel(x, weight):
    x_flat = jnp.swapaxes(x, 0, 1).reshape(-1).astype(jnp.int32)
    out5 = _lookup(x_flat, weight)
    return out5.transpose(2, 4, 0, 1, 3).reshape(BATCH, HIST, EMBEDDING_DIM)
